# baseline (speedup 1.0000x reference)
import functools

import jax
import jax.numpy as jnp
from jax import lax
from jax.experimental import pallas as pl
from jax.experimental.pallas import tpu as pltpu
from jax.experimental.pallas import tpu_sc as plsc

N = 1000
NP = 1024
H = 4
D = 64
E = 100
EP = 128
HID = H * D
L = 5
NSUB = 16
SLAB = HID // NSUB


def _dot(a, b):
    return jnp.dot(a, b, preferred_element_type=jnp.float32)


def _lrelu(x):
    return jnp.where(x >= 0, x, 0.2 * x)


def _tc_layer(first, prev, W, As, Ad, b, erow, ecol):
    def body(prev_ref, W_ref, As_ref, Ad_ref, b_ref, erow_ref, ecol_ref,
             h_ref, sp_ref, we_ref):
        srow = erow_ref[0:1, :]
        drow = erow_ref[1:2, :]
        vrow = erow_ref[2:3, :]
        dcol = ecol_ref[:, 1:2]

        n_iota = lax.broadcasted_iota(
            jnp.int32, (NP, EP), 0).astype(jnp.float32)
        s_src = jnp.where(n_iota == srow, 1.0, 0.0)
        s_dstg = jnp.where(n_iota == drow, 1.0, 0.0)
        e_iota = lax.broadcasted_iota(
            jnp.int32, (EP, NP), 1).astype(jnp.float32)
        s_dstT = jnp.where(e_iota == dcol, 1.0, 0.0)

        r_iota = lax.broadcasted_iota(jnp.int32, (HID, 8), 0)
        h_iota = lax.broadcasted_iota(jnp.int32, (HID, 8), 1)
        rexp = jnp.where(r_iota // D == h_iota, 1.0, 0.0)
        c_iota = lax.broadcasted_iota(jnp.int32, (8, HID), 1)
        g_iota = lax.broadcasted_iota(jnp.int32, (8, HID), 0)
        asel = jnp.where(c_iota // D == g_iota, 1.0, 0.0)

        xT = prev_ref[...]
        if not first:
            xT = jnp.where(xT > 0, xT, jnp.exp(jnp.minimum(xT, 0.0)) - 1.0)
        hT = _dot(W_ref[...], xT)
        al_s = _dot(asel, As_ref[...] * hT)
        al_d = _dot(asel, Ad_ref[...] * hT)
        self_a = _lrelu(al_s + al_d)

        ase = _dot(al_s, s_src)
        ade = _dot(al_d, s_dstg)
        ae = _lrelu(ase + ade)
        ce = _dot(self_a, s_dstg)
        ee = jnp.exp(ae - ce) * vrow

        s = _dot(ee, s_dstT) + (1.0 + 1e-16)
        inv_s = 1.0 / s
        se = _dot(s, s_dstg)
        we = ee / (se + (1.0 - vrow))

        h_ref[...] = hT
        sp_ref[...] = _dot(rexp, inv_s) * hT + b_ref[...]
        we_ref[...] = _dot(rexp, we)

    return pl.pallas_call(
        body,
        out_shape=(
            jax.ShapeDtypeStruct((HID, NP), jnp.float32),
            jax.ShapeDtypeStruct((HID, NP), jnp.float32),
            jax.ShapeDtypeStruct((HID, EP), jnp.float32),
        ),
    )(prev, W, As, Ad, b, erow, ecol)


def _sc_layer_body(h_hbm, sp_hbm, we_hbm, src_hbm, dst_hbm, out_hbm,
                   acc, hs, ws, sidx, didx, sem):
    cid = lax.axis_index("c")
    sid = lax.axis_index("s")

    @pl.when(cid == 0)
    def _():
        base = sid * (SLAB * NP)
        copies = [
            pltpu.async_copy(sp_hbm.at[pl.ds(base, SLAB * NP)], acc, sem),
            pltpu.async_copy(h_hbm.at[pl.ds(base, SLAB * NP)], hs, sem),
            pltpu.async_copy(
                we_hbm.at[pl.ds(sid * (SLAB * EP), SLAB * EP)], ws, sem),
            pltpu.async_copy(src_hbm, sidx, sem),
            pltpu.async_copy(dst_hbm, didx, sem),
        ]
        for c in copies:
            c.wait()

        f_iota = lax.iota(jnp.int32, 16)

        def edge_body(i, carry):
            for j in range(2):
                e_splat = jnp.full((16,), i * 2 + j, jnp.int32)
                srcv = plsc.load_gather(sidx, [e_splat])
                dstv = plsc.load_gather(didx, [e_splat])
                wv = plsc.load_gather(ws, [f_iota * EP + e_splat])
                hv = plsc.load_gather(hs, [f_iota * NP + srcv])
                plsc.addupdate_scatter(acc, [f_iota * NP + dstv], hv * wv)
            return carry

        lax.fori_loop(0, EP // 2, edge_body, 0)
        pltpu.sync_copy(acc, out_hbm.at[pl.ds(base, SLAB * NP)])


_SC_MESH = plsc.VectorSubcoreMesh(core_axis_name="c", subcore_axis_name="s")

_sc_layer = functools.partial(
    pl.kernel,
    mesh=_SC_MESH,
    compiler_params=pltpu.CompilerParams(needs_layout_passes=False),
    out_type=jax.ShapeDtypeStruct((HID * NP,), jnp.float32),
    scratch_types=[
        pltpu.VMEM((SLAB * NP,), jnp.float32),
        pltpu.VMEM((SLAB * NP,), jnp.float32),
        pltpu.VMEM((SLAB * EP,), jnp.float32),
        pltpu.VMEM((EP,), jnp.int32),
        pltpu.VMEM((EP,), jnp.int32),
        pltpu.SemaphoreType.DMA,
    ],
)(_sc_layer_body)


def kernel(L_x_, L_edge_index_, L_self_modules_convs_modules_0_modules_lin_parameters_weight_, L_self_modules_convs_modules_0_parameters_att_src_, L_self_modules_convs_modules_0_parameters_att_dst_, L_self_modules_convs_modules_0_parameters_bias_, L_self_modules_convs_modules_1_modules_lin_parameters_weight_, L_self_modules_convs_modules_1_parameters_att_src_, L_self_modules_convs_modules_1_parameters_att_dst_, L_self_modules_convs_modules_1_parameters_bias_, L_self_modules_convs_modules_2_modules_lin_parameters_weight_, L_self_modules_convs_modules_2_parameters_att_src_, L_self_modules_convs_modules_2_parameters_att_dst_, L_self_modules_convs_modules_2_parameters_bias_, L_self_modules_convs_modules_3_modules_lin_parameters_weight_, L_self_modules_convs_modules_3_parameters_att_src_, L_self_modules_convs_modules_3_parameters_att_dst_, L_self_modules_convs_modules_3_parameters_bias_, L_self_modules_convs_modules_4_modules_lin_parameters_weight_, L_self_modules_convs_modules_4_parameters_att_src_, L_self_modules_convs_modules_4_parameters_att_dst_, L_self_modules_convs_modules_4_parameters_bias_):
    kw = dict(locals())
    x = kw['L_x_']
    ei = kw['L_edge_index_']
    src_i = ei[0]
    dst_i = ei[1]
    valid_b = src_i != dst_i
    pad = EP - E

    src_f = jnp.pad(src_i.astype(jnp.float32), (0, pad), constant_values=-1.0)
    dst_f = jnp.pad(dst_i.astype(jnp.float32), (0, pad), constant_values=-1.0)
    val_f = jnp.pad(valid_b.astype(jnp.float32), (0, pad))
    erow = jnp.zeros((8, EP), jnp.float32)
    erow = erow.at[0].set(src_f).at[1].set(dst_f).at[2].set(val_f)
    ecol = jnp.zeros((EP, 8), jnp.float32)
    ecol = ecol.at[:, 0].set(src_f).at[:, 1].set(dst_f).at[:, 2].set(val_f)

    srcg = jnp.pad(src_i, (0, pad)).astype(jnp.int32)
    dstg = jnp.pad(jnp.where(valid_b, dst_i, N), (0, pad),
                   constant_values=N).astype(jnp.int32)

    xT = jnp.zeros((HID, NP), jnp.float32).at[:, :N].set(x.T)

    prev = xT
    for li in range(L):
        W = kw['L_self_modules_convs_modules_%d_modules_lin_parameters_weight_' % li]
        a_s = kw['L_self_modules_convs_modules_%d_parameters_att_src_' % li]
        a_d = kw['L_self_modules_convs_modules_%d_parameters_att_dst_' % li]
        b = kw['L_self_modules_convs_modules_%d_parameters_bias_' % li]
        hT, spT, weT = _tc_layer(li == 0, prev, W,
                                 a_s.reshape(HID, 1), a_d.reshape(HID, 1),
                                 b.reshape(HID, 1), erow, ecol)
        outT_f = _sc_layer(hT.reshape(HID * NP), spT.reshape(HID * NP),
                           weT.reshape(HID * EP), srcg, dstg)
        prev = outT_f.reshape(HID, NP)
    return prev[:, :N].T


# speedup vs baseline: 1.0004x; 1.0004x over previous
import functools

import jax
import jax.numpy as jnp
from jax import lax
from jax.experimental import pallas as pl
from jax.experimental.pallas import tpu as pltpu
from jax.experimental.pallas import tpu_sc as plsc

N = 1000
NP = 1024
H = 4
D = 64
E = 100
EP = 128
HID = H * D
L = 5
NSUB = 16
SLAB = HID // NSUB


def _dot(a, b):
    return jnp.dot(a, b, preferred_element_type=jnp.float32)


def _lrelu(x):
    return jnp.where(x >= 0, x, 0.2 * x)


def _tc_layer(first, prev, W, As, Ad, b, erow, ecol):
    def body(prev_ref, W_ref, As_ref, Ad_ref, b_ref, erow_ref, ecol_ref,
             h_ref, sp_ref, we_ref):
        srow = erow_ref[0:1, :]
        drow = erow_ref[1:2, :]
        vrow = erow_ref[2:3, :]
        dcol = ecol_ref[:, 1:2]

        n_iota = lax.broadcasted_iota(
            jnp.int32, (NP, EP), 0).astype(jnp.float32)
        s_src = jnp.where(n_iota == srow, 1.0, 0.0)
        s_dstg = jnp.where(n_iota == drow, 1.0, 0.0)
        e_iota = lax.broadcasted_iota(
            jnp.int32, (EP, NP), 1).astype(jnp.float32)
        s_dstT = jnp.where(e_iota == dcol, 1.0, 0.0)

        r_iota = lax.broadcasted_iota(jnp.int32, (HID, 8), 0)
        h_iota = lax.broadcasted_iota(jnp.int32, (HID, 8), 1)
        rexp = jnp.where(r_iota // D == h_iota, 1.0, 0.0)
        c_iota = lax.broadcasted_iota(jnp.int32, (8, HID), 1)
        g_iota = lax.broadcasted_iota(jnp.int32, (8, HID), 0)
        asel = jnp.where(c_iota // D == g_iota, 1.0, 0.0)

        xT = prev_ref[...]
        if not first:
            xT = jnp.where(xT > 0, xT, jnp.exp(jnp.minimum(xT, 0.0)) - 1.0)
        hT = _dot(W_ref[...], xT)
        al_s = _dot(asel, As_ref[...] * hT)
        al_d = _dot(asel, Ad_ref[...] * hT)
        self_a = _lrelu(al_s + al_d)

        ase = _dot(al_s, s_src)
        ade = _dot(al_d, s_dstg)
        ae = _lrelu(ase + ade)
        ce = _dot(self_a, s_dstg)
        ee = jnp.exp(ae - ce) * vrow

        s = _dot(ee, s_dstT) + (1.0 + 1e-16)
        inv_s = 1.0 / s
        se = _dot(s, s_dstg)
        we = ee / (se + (1.0 - vrow))

        h_ref[...] = hT
        sp_ref[...] = _dot(rexp, inv_s) * hT + b_ref[...]
        we_ref[...] = _dot(rexp, we)

    return pl.pallas_call(
        body,
        compiler_params=pltpu.CompilerParams(skip_device_barrier=True),
        out_shape=(
            jax.ShapeDtypeStruct((HID, NP), jnp.float32),
            jax.ShapeDtypeStruct((HID, NP), jnp.float32),
            jax.ShapeDtypeStruct((HID, EP), jnp.float32),
        ),
    )(prev, W, As, Ad, b, erow, ecol)


def _sc_layer_body(h_hbm, sp_hbm, we_hbm, src_hbm, dst_hbm, out_hbm,
                   acc, hs, ws, sidx, didx, sem):
    cid = lax.axis_index("c")
    sid = lax.axis_index("s")

    @pl.when(cid == 0)
    def _():
        base = sid * (SLAB * NP)
        copies = [
            pltpu.async_copy(sp_hbm.at[pl.ds(base, SLAB * NP)], acc, sem),
            pltpu.async_copy(h_hbm.at[pl.ds(base, SLAB * NP)], hs, sem),
            pltpu.async_copy(
                we_hbm.at[pl.ds(sid * (SLAB * EP), SLAB * EP)], ws, sem),
            pltpu.async_copy(src_hbm, sidx, sem),
            pltpu.async_copy(dst_hbm, didx, sem),
        ]
        for c in copies:
            c.wait()

        f_iota = lax.iota(jnp.int32, 16)

        def edge_body(i, carry):
            for j in range(2):
                e_splat = jnp.full((16,), i * 2 + j, jnp.int32)
                srcv = plsc.load_gather(sidx, [e_splat])
                dstv = plsc.load_gather(didx, [e_splat])
                wv = plsc.load_gather(ws, [f_iota * EP + e_splat])
                hv = plsc.load_gather(hs, [f_iota * NP + srcv])
                plsc.addupdate_scatter(acc, [f_iota * NP + dstv], hv * wv)
            return carry

        lax.fori_loop(0, EP // 2, edge_body, 0)
        pltpu.sync_copy(acc, out_hbm.at[pl.ds(base, SLAB * NP)])


_SC_MESH = plsc.VectorSubcoreMesh(core_axis_name="c", subcore_axis_name="s")

_sc_layer = functools.partial(
    pl.kernel,
    mesh=_SC_MESH,
    compiler_params=pltpu.CompilerParams(needs_layout_passes=False),
    out_type=jax.ShapeDtypeStruct((HID * NP,), jnp.float32),
    scratch_types=[
        pltpu.VMEM((SLAB * NP,), jnp.float32),
        pltpu.VMEM((SLAB * NP,), jnp.float32),
        pltpu.VMEM((SLAB * EP,), jnp.float32),
        pltpu.VMEM((EP,), jnp.int32),
        pltpu.VMEM((EP,), jnp.int32),
        pltpu.SemaphoreType.DMA,
    ],
)(_sc_layer_body)


def kernel(L_x_, L_edge_index_, L_self_modules_convs_modules_0_modules_lin_parameters_weight_, L_self_modules_convs_modules_0_parameters_att_src_, L_self_modules_convs_modules_0_parameters_att_dst_, L_self_modules_convs_modules_0_parameters_bias_, L_self_modules_convs_modules_1_modules_lin_parameters_weight_, L_self_modules_convs_modules_1_parameters_att_src_, L_self_modules_convs_modules_1_parameters_att_dst_, L_self_modules_convs_modules_1_parameters_bias_, L_self_modules_convs_modules_2_modules_lin_parameters_weight_, L_self_modules_convs_modules_2_parameters_att_src_, L_self_modules_convs_modules_2_parameters_att_dst_, L_self_modules_convs_modules_2_parameters_bias_, L_self_modules_convs_modules_3_modules_lin_parameters_weight_, L_self_modules_convs_modules_3_parameters_att_src_, L_self_modules_convs_modules_3_parameters_att_dst_, L_self_modules_convs_modules_3_parameters_bias_, L_self_modules_convs_modules_4_modules_lin_parameters_weight_, L_self_modules_convs_modules_4_parameters_att_src_, L_self_modules_convs_modules_4_parameters_att_dst_, L_self_modules_convs_modules_4_parameters_bias_):
    kw = dict(locals())
    x = kw['L_x_']
    ei = kw['L_edge_index_']
    src_i = ei[0]
    dst_i = ei[1]
    valid_b = src_i != dst_i
    pad = EP - E

    src_f = jnp.pad(src_i.astype(jnp.float32), (0, pad), constant_values=-1.0)
    dst_f = jnp.pad(dst_i.astype(jnp.float32), (0, pad), constant_values=-1.0)
    val_f = jnp.pad(valid_b.astype(jnp.float32), (0, pad))
    erow = jnp.zeros((8, EP), jnp.float32)
    erow = erow.at[0].set(src_f).at[1].set(dst_f).at[2].set(val_f)
    ecol = jnp.zeros((EP, 8), jnp.float32)
    ecol = ecol.at[:, 0].set(src_f).at[:, 1].set(dst_f).at[:, 2].set(val_f)

    srcg = jnp.pad(src_i, (0, pad)).astype(jnp.int32)
    dstg = jnp.pad(jnp.where(valid_b, dst_i, N), (0, pad),
                   constant_values=N).astype(jnp.int32)

    xT = jnp.zeros((HID, NP), jnp.float32).at[:, :N].set(x.T)

    prev = xT
    for li in range(L):
        W = kw['L_self_modules_convs_modules_%d_modules_lin_parameters_weight_' % li]
        a_s = kw['L_self_modules_convs_modules_%d_parameters_att_src_' % li]
        a_d = kw['L_self_modules_convs_modules_%d_parameters_att_dst_' % li]
        b = kw['L_self_modules_convs_modules_%d_parameters_bias_' % li]
        hT, spT, weT = _tc_layer(li == 0, prev, W,
                                 a_s.reshape(HID, 1), a_d.reshape(HID, 1),
                                 b.reshape(HID, 1), erow, ecol)
        outT_f = _sc_layer(hT.reshape(HID * NP), spT.reshape(HID * NP),
                           weT.reshape(HID * EP), srcg, dstg)
        prev = outT_f.reshape(HID, NP)
    return prev[:, :N].T


# speedup vs baseline: 1.0019x; 1.0015x over previous
import functools

import jax
import jax.numpy as jnp
from jax import lax
from jax.experimental import pallas as pl
from jax.experimental.pallas import tpu as pltpu
from jax.experimental.pallas import tpu_sc as plsc

N = 1000
NP = 1024
H = 4
D = 64
E = 100
EP = 128
HID = H * D
L = 5
NSUB = 16
SLAB = HID // NSUB


def _dot(a, b):
    return jnp.dot(a, b, preferred_element_type=jnp.float32)


def _lrelu(x):
    return jnp.where(x >= 0, x, 0.2 * x)


def _tc_layer(first, prev, W, As, Ad, b, erow, ecol):
    def body(prev_ref, W_ref, As_ref, Ad_ref, b_ref, erow_ref, ecol_ref,
             h_ref, sp_ref, we_ref):
        srow = erow_ref[0:1, :]
        drow = erow_ref[1:2, :]
        vrow = erow_ref[2:3, :]
        dcol = ecol_ref[:, 1:2]

        n_iota = lax.broadcasted_iota(
            jnp.int32, (NP, EP), 0).astype(jnp.float32)
        s_src = jnp.where(n_iota == srow, 1.0, 0.0)
        s_dstg = jnp.where(n_iota == drow, 1.0, 0.0)
        e_iota = lax.broadcasted_iota(
            jnp.int32, (EP, NP), 1).astype(jnp.float32)
        s_dstT = jnp.where(e_iota == dcol, 1.0, 0.0)

        r_iota = lax.broadcasted_iota(jnp.int32, (HID, 8), 0)
        h_iota = lax.broadcasted_iota(jnp.int32, (HID, 8), 1)
        rexp = jnp.where(r_iota // D == h_iota, 1.0, 0.0)
        c_iota = lax.broadcasted_iota(jnp.int32, (8, HID), 1)
        g_iota = lax.broadcasted_iota(jnp.int32, (8, HID), 0)
        asel = jnp.where(c_iota // D == g_iota, 1.0, 0.0)

        xT = prev_ref[...]
        if not first:
            xT = jnp.where(xT > 0, xT, jnp.exp(jnp.minimum(xT, 0.0)) - 1.0)
        hT = _dot(W_ref[...], xT)
        al_s = _dot(asel, As_ref[...] * hT)
        al_d = _dot(asel, Ad_ref[...] * hT)
        self_a = _lrelu(al_s + al_d)

        ase = _dot(al_s, s_src)
        ade = _dot(al_d, s_dstg)
        ae = _lrelu(ase + ade)
        ce = _dot(self_a, s_dstg)
        ee = jnp.exp(ae - ce) * vrow

        s = _dot(ee, s_dstT) + (1.0 + 1e-16)
        inv_s = 1.0 / s
        se = _dot(s, s_dstg)
        we = ee / (se + (1.0 - vrow))

        h_ref[...] = hT
        sp_ref[...] = _dot(rexp, inv_s) * hT + b_ref[...]
        we_ref[...] = _dot(rexp, we)

    return pl.pallas_call(
        body,
        out_shape=(
            jax.ShapeDtypeStruct((HID, NP), jnp.float32),
            jax.ShapeDtypeStruct((HID, NP), jnp.float32),
            jax.ShapeDtypeStruct((HID, EP), jnp.float32),
        ),
    )(prev, W, As, Ad, b, erow, ecol)


def _sc_layer_body(h_hbm, sp_hbm, we_hbm, src_hbm, dst_hbm, out_hbm,
                   acc, hs, ws, sidx, didx, sem):
    cid = lax.axis_index("c")
    sid = lax.axis_index("s")

    @pl.when(cid == 0)
    def _():
        base = sid * (SLAB * NP)
        copies = [
            pltpu.async_copy(sp_hbm.at[pl.ds(base, SLAB * NP)], acc, sem),
            pltpu.async_copy(h_hbm.at[pl.ds(base, SLAB * NP)], hs, sem),
            pltpu.async_copy(
                we_hbm.at[pl.ds(sid * (SLAB * EP), SLAB * EP)], ws, sem),
            pltpu.async_copy(src_hbm, sidx, sem),
            pltpu.async_copy(dst_hbm, didx, sem),
        ]
        for c in copies:
            c.wait()

        f_iota = lax.iota(jnp.int32, 16)

        def edge_body(i, carry):
            for j in range(2):
                e_splat = jnp.full((16,), i * 2 + j, jnp.int32)
                srcv = plsc.load_gather(sidx, [e_splat])
                dstv = plsc.load_gather(didx, [e_splat])
                wv = plsc.load_gather(ws, [f_iota * EP + e_splat])
                hv = plsc.load_gather(hs, [f_iota * NP + srcv])
                plsc.addupdate_scatter(acc, [f_iota * NP + dstv], hv * wv)
            return carry

        lax.fori_loop(0, EP // 2, edge_body, 0)
        pltpu.sync_copy(acc, out_hbm.at[pl.ds(base, SLAB * NP)])


_SC_MESH = plsc.VectorSubcoreMesh(core_axis_name="c", subcore_axis_name="s")

_sc_layer = functools.partial(
    pl.kernel,
    mesh=_SC_MESH,
    compiler_params=pltpu.CompilerParams(needs_layout_passes=False),
    out_type=jax.ShapeDtypeStruct((HID * NP,), jnp.float32),
    scratch_types=[
        pltpu.VMEM((SLAB * NP,), jnp.float32),
        pltpu.VMEM((SLAB * NP,), jnp.float32),
        pltpu.VMEM((SLAB * EP,), jnp.float32),
        pltpu.VMEM((EP,), jnp.int32),
        pltpu.VMEM((EP,), jnp.int32),
        pltpu.SemaphoreType.DMA,
    ],
)(_sc_layer_body)


def kernel(L_x_, L_edge_index_, L_self_modules_convs_modules_0_modules_lin_parameters_weight_, L_self_modules_convs_modules_0_parameters_att_src_, L_self_modules_convs_modules_0_parameters_att_dst_, L_self_modules_convs_modules_0_parameters_bias_, L_self_modules_convs_modules_1_modules_lin_parameters_weight_, L_self_modules_convs_modules_1_parameters_att_src_, L_self_modules_convs_modules_1_parameters_att_dst_, L_self_modules_convs_modules_1_parameters_bias_, L_self_modules_convs_modules_2_modules_lin_parameters_weight_, L_self_modules_convs_modules_2_parameters_att_src_, L_self_modules_convs_modules_2_parameters_att_dst_, L_self_modules_convs_modules_2_parameters_bias_, L_self_modules_convs_modules_3_modules_lin_parameters_weight_, L_self_modules_convs_modules_3_parameters_att_src_, L_self_modules_convs_modules_3_parameters_att_dst_, L_self_modules_convs_modules_3_parameters_bias_, L_self_modules_convs_modules_4_modules_lin_parameters_weight_, L_self_modules_convs_modules_4_parameters_att_src_, L_self_modules_convs_modules_4_parameters_att_dst_, L_self_modules_convs_modules_4_parameters_bias_):
    kw = dict(locals())
    x = kw['L_x_']
    ei = kw['L_edge_index_']
    src_i = ei[0]
    dst_i = ei[1]
    valid_b = src_i != dst_i
    pad = EP - E

    src_f = jnp.pad(src_i.astype(jnp.float32), (0, pad), constant_values=-1.0)
    dst_f = jnp.pad(dst_i.astype(jnp.float32), (0, pad), constant_values=-1.0)
    val_f = jnp.pad(valid_b.astype(jnp.float32), (0, pad))
    erow = jnp.zeros((8, EP), jnp.float32)
    erow = erow.at[0].set(src_f).at[1].set(dst_f).at[2].set(val_f)
    ecol = jnp.zeros((EP, 8), jnp.float32)
    ecol = ecol.at[:, 0].set(src_f).at[:, 1].set(dst_f).at[:, 2].set(val_f)

    srcg = jnp.pad(src_i, (0, pad)).astype(jnp.int32)
    dstg = jnp.pad(jnp.where(valid_b, dst_i, N), (0, pad),
                   constant_values=N).astype(jnp.int32)

    xT = jnp.zeros((HID, NP), jnp.float32).at[:, :N].set(x.T)

    prev = xT
    for li in range(L):
        W = kw['L_self_modules_convs_modules_%d_modules_lin_parameters_weight_' % li]
        a_s = kw['L_self_modules_convs_modules_%d_parameters_att_src_' % li]
        a_d = kw['L_self_modules_convs_modules_%d_parameters_att_dst_' % li]
        b = kw['L_self_modules_convs_modules_%d_parameters_bias_' % li]
        hT, spT, weT = _tc_layer(li == 0, prev, W,
                                 a_s.reshape(HID, 1), a_d.reshape(HID, 1),
                                 b.reshape(HID, 1), erow, ecol)
        outT_f = _sc_layer(hT.reshape(HID * NP), spT.reshape(HID * NP),
                           weT.reshape(HID * EP), srcg, dstg)
        prev = outT_f.reshape(HID, NP)
    return prev[:, :N].T
